# baseline (speedup 1.0000x reference)
import functools

import jax
import jax.numpy as jnp
from jax import lax
from jax.experimental import pallas as pl
from jax.experimental.pallas import tpu as pltpu
from jax.experimental.pallas import tpu_sc as plsc

N_CODES = 8192
DIM = 256
FLAT = 16384
RM = 256
TILES = FLAT // RM
TOTAL = FLAT * DIM


def _argmin_body(x_ref, et_ref, sx_ref, se_ref,
                 idx_ref, loss_ref, perp_ref,
                 counts_ref, acc_ref):
    i = pl.program_id(0)
    x = x_ref[...]
    et = et_ref[...]
    m = lax.dot_general(x, et, (((1,), (0,)), ((), ())),
                        preferred_element_type=jnp.float32)
    d = se_ref[...] + m
    mind2 = jnp.min(d, axis=1, keepdims=True)
    mind = sx_ref[...] + mind2
    col = lax.broadcasted_iota(jnp.int32, (RM, N_CODES), 1)
    idx = jnp.min(jnp.where(d == mind2, col, N_CODES), axis=1)
    idx_ref[0, 0, :] = idx

    part = jnp.sum(jnp.where(idx[:, None] == col, 1.0, 0.0),
                   axis=0, keepdims=True)
    s = jnp.sum(mind)

    @pl.when(i == 0)
    def _():
        counts_ref[...] = part
        acc_ref[0] = s

    @pl.when(i > 0)
    def _():
        counts_ref[...] += part
        acc_ref[0] += s

    @pl.when(i == TILES - 1)
    def _():
        loss_ref[0, 0] = 1.25 * (acc_ref[0] / TOTAL)
        p = counts_ref[...] * (1.0 / FLAT)
        ent = -jnp.sum(p * jnp.log(p + 1e-10))
        perp_ref[0, 0] = jnp.exp(ent)


_argmin_call = pl.pallas_call(
    _argmin_body,
    grid=(TILES,),
    in_specs=[
        pl.BlockSpec((RM, DIM), lambda i: (i, 0)),
        pl.BlockSpec((DIM, N_CODES), lambda i: (0, 0)),
        pl.BlockSpec((RM, 1), lambda i: (i, 0)),
        pl.BlockSpec((1, N_CODES), lambda i: (0, 0)),
    ],
    out_specs=[
        pl.BlockSpec((1, 1, RM), lambda i: (i, 0, 0)),
        pl.BlockSpec(memory_space=pltpu.SMEM),
        pl.BlockSpec(memory_space=pltpu.SMEM),
    ],
    out_shape=[
        jax.ShapeDtypeStruct((TILES, 1, RM), jnp.int32),
        jax.ShapeDtypeStruct((1, 1), jnp.float32),
        jax.ShapeDtypeStruct((1, 1), jnp.float32),
    ],
    scratch_shapes=[
        pltpu.VMEM((1, N_CODES), jnp.float32),
        pltpu.SMEM((1,), jnp.float32),
    ],
    compiler_params=pltpu.CompilerParams(
        dimension_semantics=("arbitrary",)),
)


_NC, _NS = 2, 16
_NW = _NC * _NS
_BPW = FLAT // _NW
_CHUNK = 128


def _sc_gather_body(table_hbm, idx_hbm, out_hbm, idx_v, rows0, rows1,
                    sem0, sem1):
    wid = lax.axis_index("s") * _NC + lax.axis_index("c")
    base = wid * _BPW
    pltpu.sync_copy(idx_hbm.at[pl.ds(base, _BPW)], idx_v)
    bufs, sems = (rows0, rows1), (sem0, sem1)
    nch = _BPW // _CHUNK

    def start(c):
        return pltpu.async_copy(
            table_hbm.at[idx_v.at[pl.ds(c * _CHUNK, _CHUNK)]],
            bufs[c % 2], sems[c % 2])

    cps = [None] * nch
    cps[0], cps[1] = start(0), start(1)
    for c in range(nch):
        cps[c].wait()
        pltpu.sync_copy(bufs[c % 2],
                        out_hbm.at[pl.ds(base + c * _CHUNK, _CHUNK)])
        if c + 2 < nch:
            cps[c + 2] = start(c + 2)


@functools.cache
def _sc_gather():
    return pl.kernel(
        _sc_gather_body,
        out_type=jax.ShapeDtypeStruct((FLAT, DIM), jnp.float32),
        mesh=plsc.VectorSubcoreMesh(core_axis_name="c", subcore_axis_name="s"),
        scratch_types=[
            pltpu.VMEM((_BPW,), jnp.int32),
            pltpu.VMEM((_CHUNK, DIM), jnp.float32),
            pltpu.VMEM((_CHUNK, DIM), jnp.float32),
            pltpu.SemaphoreType.DMA,
            pltpu.SemaphoreType.DMA,
        ],
    )


def kernel(inputs, embedding):
    flat = inputs.reshape(-1, DIM)
    sx = jnp.sum(flat ** 2, axis=1, keepdims=True)
    se = jnp.sum(embedding ** 2, axis=1)[None, :]
    et = -2.0 * embedding.T
    idx3, loss, perp = _argmin_call(flat, et, sx, se)
    idx = idx3.reshape(FLAT)
    quantized = _sc_gather()(embedding, idx)
    return quantized.reshape(inputs.shape), loss[0, 0], perp[0, 0]


# speedup vs baseline: 1.0567x; 1.0567x over previous
import functools

import jax
import jax.numpy as jnp
from jax import lax
from jax.experimental import pallas as pl
from jax.experimental.pallas import tpu as pltpu
from jax.experimental.pallas import tpu_sc as plsc

N_CODES = 8192
DIM = 256
FLAT = 16384
RM = 256
TILES = FLAT // RM
TOTAL = FLAT * DIM


def _argmin_body(x_ref, et_ref, sx_ref, se_ref,
                 idx_ref, loss_ref, perp_ref,
                 counts_ref, acc_ref):
    i = pl.program_id(0)
    x = x_ref[...]
    et = et_ref[...]
    m = lax.dot_general(x, et, (((1,), (0,)), ((), ())),
                        preferred_element_type=jnp.float32)
    d = (sx_ref[...] + se_ref[...]) - 2.0 * m
    mind = jnp.min(d, axis=1, keepdims=True)
    col = lax.broadcasted_iota(jnp.int32, (RM, N_CODES), 1)
    idx = jnp.min(jnp.where(d == mind, col, N_CODES), axis=1)
    idx_ref[0, 0, :] = idx

    part = jnp.sum(jnp.where(idx[:, None] == col, 1.0, 0.0),
                   axis=0, keepdims=True)
    s = jnp.sum(mind)

    @pl.when(i == 0)
    def _():
        counts_ref[...] = part
        acc_ref[0] = s

    @pl.when(i > 0)
    def _():
        counts_ref[...] += part
        acc_ref[0] += s

    @pl.when(i == TILES - 1)
    def _():
        loss_ref[0, 0] = 1.25 * (acc_ref[0] / TOTAL)
        p = counts_ref[...] * (1.0 / FLAT)
        ent = -jnp.sum(p * jnp.log(p + 1e-10))
        perp_ref[0, 0] = jnp.exp(ent)


_argmin_call = pl.pallas_call(
    _argmin_body,
    grid=(TILES,),
    in_specs=[
        pl.BlockSpec((RM, DIM), lambda i: (i, 0)),
        pl.BlockSpec((DIM, N_CODES), lambda i: (0, 0)),
        pl.BlockSpec((RM, 1), lambda i: (i, 0)),
        pl.BlockSpec((1, N_CODES), lambda i: (0, 0)),
    ],
    out_specs=[
        pl.BlockSpec((1, 1, RM), lambda i: (i, 0, 0)),
        pl.BlockSpec(memory_space=pltpu.SMEM),
        pl.BlockSpec(memory_space=pltpu.SMEM),
    ],
    out_shape=[
        jax.ShapeDtypeStruct((TILES, 1, RM), jnp.int32),
        jax.ShapeDtypeStruct((1, 1), jnp.float32),
        jax.ShapeDtypeStruct((1, 1), jnp.float32),
    ],
    scratch_shapes=[
        pltpu.VMEM((1, N_CODES), jnp.float32),
        pltpu.SMEM((1,), jnp.float32),
    ],
    compiler_params=pltpu.CompilerParams(
        dimension_semantics=("arbitrary",)),
)


_NC, _NS = 2, 16
_NW = _NC * _NS
_BPW = FLAT // _NW
_CHUNK = 128


def _sc_gather_body(table_hbm, idx_hbm, out_hbm, idx_v, rows0, rows1,
                    sem0, sem1):
    wid = lax.axis_index("s") * _NC + lax.axis_index("c")
    base = wid * _BPW
    pltpu.sync_copy(idx_hbm.at[pl.ds(base, _BPW)], idx_v)
    bufs, sems = (rows0, rows1), (sem0, sem1)
    nch = _BPW // _CHUNK

    def start(c):
        return pltpu.async_copy(
            table_hbm.at[idx_v.at[pl.ds(c * _CHUNK, _CHUNK)]],
            bufs[c % 2], sems[c % 2])

    cps = [None] * nch
    cps[0], cps[1] = start(0), start(1)
    for c in range(nch):
        cps[c].wait()
        pltpu.sync_copy(bufs[c % 2],
                        out_hbm.at[pl.ds(base + c * _CHUNK, _CHUNK)])
        if c + 2 < nch:
            cps[c + 2] = start(c + 2)


@functools.cache
def _sc_gather():
    return pl.kernel(
        _sc_gather_body,
        out_type=jax.ShapeDtypeStruct((FLAT, DIM), jnp.float32),
        mesh=plsc.VectorSubcoreMesh(core_axis_name="c", subcore_axis_name="s"),
        scratch_types=[
            pltpu.VMEM((_BPW,), jnp.int32),
            pltpu.VMEM((_CHUNK, DIM), jnp.float32),
            pltpu.VMEM((_CHUNK, DIM), jnp.float32),
            pltpu.SemaphoreType.DMA,
            pltpu.SemaphoreType.DMA,
        ],
    )


def kernel(inputs, embedding):
    flat = inputs.reshape(-1, DIM)
    sx = jnp.sum(flat ** 2, axis=1, keepdims=True)
    se = jnp.sum(embedding ** 2, axis=1)[None, :]
    et = embedding.T
    idx3, loss, perp = _argmin_call(flat, et, sx, se)
    idx = idx3.reshape(FLAT)
    quantized = _sc_gather()(embedding, idx)
    return quantized.reshape(inputs.shape), loss[0, 0], perp[0, 0]
